# baseline (speedup 1.0000x reference)
import functools

import jax
import jax.numpy as jnp
from jax import lax
from jax.experimental import pallas as pl
from jax.experimental.pallas import tpu as pltpu
from jax.experimental.pallas import tpu_sc as plsc

_NC = 2
_NS = 16
_NW = _NC * _NS
_W = 256
_CH = 256


def kernel(x, y):
    del y
    B, S, D = x.shape
    R = B * S
    rpw = R // _NW
    x2 = x.reshape(R, D)
    mesh = plsc.VectorSubcoreMesh(core_axis_name="c", subcore_axis_name="s")

    @functools.partial(
        pl.kernel,
        mesh=mesh,
        out_type=jax.ShapeDtypeStruct((R, D), jnp.float32),
        scratch_types=[
            pltpu.VMEM((_CH, 128), jnp.float32),
            pltpu.VMEM((_CH, _W), jnp.float32),
        ],
    )
    def sc_fill(x_hbm, out_hbm, vals_v, tmpl_v):
        wid = lax.axis_index("s") * _NC + lax.axis_index("c")
        base = wid * rpw
        for c in range(rpw // _CH):
            cbase = base + c * _CH
            pltpu.sync_copy(x_hbm.at[pl.ds(cbase, _CH), pl.ds(0, 128)], vals_v)

            def fill_row(r, carry):
                head = vals_v[r, pl.ds(0, 16)]
                vec = jnp.full((16,), head[1], dtype=jnp.float32)
                for j in range(_W // 16):
                    tmpl_v[r, pl.ds(j * 16, 16)] = vec
                return carry

            lax.fori_loop(0, _CH, fill_row, 0)
            for k in range(D // _W):
                pltpu.sync_copy(
                    tmpl_v, out_hbm.at[pl.ds(cbase, _CH), pl.ds(k * _W, _W)]
                )

    return sc_fill(x2).reshape(B, S, D)


# speedup vs baseline: 1.5945x; 1.5945x over previous
import jax
import jax.numpy as jnp
from jax.experimental import pallas as pl

_BS = 1024


def _bcast_kernel(x_ref, o_ref):
    col = x_ref[0, :, 1:2]
    o_ref[0] = jnp.broadcast_to(col, o_ref.shape[1:])


def kernel(x, y):
    del y
    B, S, D = x.shape
    return pl.pallas_call(
        _bcast_kernel,
        grid=(B, S // _BS),
        in_specs=[pl.BlockSpec((1, _BS, 128), lambda b, s: (b, s, 0))],
        out_specs=pl.BlockSpec((1, _BS, D), lambda b, s: (b, s, 0)),
        out_shape=jax.ShapeDtypeStruct((B, S, D), x.dtype),
    )(x)
